# Initial kernel scaffold; baseline (speedup 1.0000x reference)
import jax
import jax.numpy as jnp
from jax.experimental import pallas as pl


def kernel(x, tok_table, pos_table):
    raise NotImplementedError("write your pallas kernel here")


# speedup vs baseline: 1.2879x; 1.2879x over previous
import jax
import jax.numpy as jnp
from jax import lax
from jax.experimental import pallas as pl
from jax.experimental.pallas import tpu as pltpu
from jax.experimental.pallas import tpu_sc as plsc

B, L, D = 1024, 200, 64
NC, NS = 2, 16
NW = NC * NS
ROWS_PER_W = B * L // NW
SEQ_PER_W = ROWS_PER_W // L
G = 100
NG = ROWS_PER_W // G
GPC = L // G


def _body(x_hbm, tok_hbm, pos_hbm, out_hbm, idx_v, pos_v, buf, gsem):
    c = lax.axis_index("c")
    s = lax.axis_index("s")
    w = s * NC + c

    pltpu.sync_copy(x_hbm.at[w], idx_v)
    pltpu.sync_copy(pos_hbm, pos_v)

    def seq_step(ci, carry):
        cp0 = pltpu.async_copy(tok_hbm.at[idx_v.at[GPC * ci]],
                               buf.at[pl.ds(0, G)], gsem)
        cp1 = pltpu.async_copy(tok_hbm.at[idx_v.at[GPC * ci + 1]],
                               buf.at[pl.ds(G, G)], gsem)
        cp0.wait()
        cp1.wait()

        def add_row(r, carry2):
            for dd in range(0, D, 16):
                buf[r, pl.ds(dd, 16)] = (buf[r, pl.ds(dd, 16)]
                                         + pos_v[r, pl.ds(dd, 16)])
            return carry2

        lax.fori_loop(0, L, add_row, 0)
        pltpu.sync_copy(buf, out_hbm.at[pl.ds(w * ROWS_PER_W + ci * L, L)])
        return carry

    lax.fori_loop(0, SEQ_PER_W, seq_step, 0)


def kernel(x, tok_table, pos_table):
    x3 = x.reshape(NW, NG, G).astype(jnp.int32)
    mesh = plsc.VectorSubcoreMesh(core_axis_name="c", subcore_axis_name="s")
    run = pl.kernel(
        _body,
        out_type=jax.ShapeDtypeStruct((B * L, D), jnp.float32),
        mesh=mesh,
        compiler_params=pltpu.CompilerParams(use_tc_tiling_on_sc=False),
        scratch_types=[
            pltpu.VMEM((NG, G), jnp.int32),
            pltpu.VMEM((L, D), jnp.float32),
            pltpu.VMEM((L, D), jnp.float32),
            pltpu.SemaphoreType.DMA,
        ],
    )
    out = run(x3, tok_table, pos_table)
    return out.reshape(B, L, D)


# speedup vs baseline: 1.3340x; 1.0358x over previous
import jax
import jax.numpy as jnp
from jax import lax
from jax.experimental import pallas as pl
from jax.experimental.pallas import tpu as pltpu
from jax.experimental.pallas import tpu_sc as plsc

B, L, D = 1024, 200, 64
NC, NS = 2, 16
NW = NC * NS
ROWS_PER_W = B * L // NW
CH = 128
NCH = ROWS_PER_W // CH


def _body(x_hbm, tok_hbm, pos2_hbm, out_hbm,
          idx_v, pos_v, buf0, buf1, gsem0, gsem1, osem0, osem1):
    c = lax.axis_index("c")
    s = lax.axis_index("s")
    w = s * NC + c

    pltpu.sync_copy(x_hbm.at[w], idx_v)
    pltpu.sync_copy(pos2_hbm, pos_v)

    bufs = (buf0, buf1)
    gsems = (gsem0, gsem1)
    osems = (osem0, osem1)

    def gather(cc):
        return pltpu.async_copy(tok_hbm.at[idx_v.at[cc]],
                                bufs[cc % 2], gsems[cc % 2])

    def out_start(cc):
        return pltpu.async_copy(
            bufs[cc % 2],
            out_hbm.at[pl.ds(w * ROWS_PER_W + cc * CH, CH)],
            osems[cc % 2])

    def add_pos(cc):
        b = bufs[cc % 2]
        p = (cc * CH) % L

        def add4(k, carry):
            for j in range(4):
                r = 4 * k + j
                for dd in range(0, D, 16):
                    b[r, pl.ds(dd, 16)] = (b[r, pl.ds(dd, 16)]
                                           + pos_v[p + r, pl.ds(dd, 16)])
            return carry

        lax.fori_loop(0, CH // 4, add4, 0)

    cps_g = [None] * NCH
    cps_o = [None] * NCH
    cps_g[0] = gather(0)
    for cc in range(NCH):
        if cc + 1 < NCH:
            if cc - 1 >= 0:
                cps_o[cc - 1].wait()
            cps_g[cc + 1] = gather(cc + 1)
        cps_g[cc].wait()
        add_pos(cc)
        cps_o[cc] = out_start(cc)
    cps_o[NCH - 2].wait()
    cps_o[NCH - 1].wait()


def kernel(x, tok_table, pos_table):
    x3 = x.reshape(NW, NCH, CH).astype(jnp.int32)
    pos2 = jnp.concatenate([pos_table, pos_table], axis=0)
    mesh = plsc.VectorSubcoreMesh(core_axis_name="c", subcore_axis_name="s")
    run = pl.kernel(
        _body,
        out_type=jax.ShapeDtypeStruct((B * L, D), jnp.float32),
        mesh=mesh,
        compiler_params=pltpu.CompilerParams(use_tc_tiling_on_sc=False),
        scratch_types=[
            pltpu.VMEM((NCH, CH), jnp.int32),
            pltpu.VMEM((2 * L, D), jnp.float32),
            pltpu.VMEM((CH, D), jnp.float32),
            pltpu.VMEM((CH, D), jnp.float32),
            pltpu.SemaphoreType.DMA,
            pltpu.SemaphoreType.DMA,
            pltpu.SemaphoreType.DMA,
            pltpu.SemaphoreType.DMA,
        ],
    )
    out = run(x3, tok_table, pos2)
    return out.reshape(B, L, D)
